# baseline (speedup 1.0000x reference)
import functools

import jax
import jax.numpy as jnp
from jax import lax
from jax.experimental import pallas as pl
from jax.experimental.pallas import tpu as pltpu
from jax.experimental.pallas import tpu_sc as plsc

_UNPOOL = (0, 0, 1, 1, 2, 3, 3, 4, 5, 5, 6, 7, 7, 8, 8, 9, 9, 10, 10, 11, 11, 11)

_FANOUT = tuple(
    tuple(j for j, src in enumerate(_UNPOOL) if src == jp)
    for jp in range(_UNPOOL[-1] + 1)
)


def kernel(x_p):
    B, JP, C, F = x_p.shape
    NJ = len(_UNPOOL)
    D = C * F
    x2 = x_p.reshape(B, JP, D)

    info = plsc.get_sparse_core_info()
    NC, NS = info.num_cores, info.num_subcores
    NW = NC * NS
    bpw = B // NW

    mesh = plsc.VectorSubcoreMesh(core_axis_name="c", subcore_axis_name="s")

    @functools.partial(
        pl.kernel,
        mesh=mesh,
        out_type=jax.ShapeDtypeStruct((B, NJ, D), jnp.float32),
        scratch_types=[pltpu.VMEM((D,), jnp.float32)],
    )
    def unpool_sc(x_hbm, out_hbm, buf):
        wid = lax.axis_index("s") * NC + lax.axis_index("c")
        for bi in range(bpw):
            b = wid * bpw + bi
            for jp in range(JP):
                pltpu.sync_copy(x_hbm.at[b, jp], buf)
                for j in _FANOUT[jp]:
                    pltpu.sync_copy(buf, out_hbm.at[b, j])

    return unpool_sc(x2).reshape(B, NJ, C, F)


# speedup vs baseline: 1.0638x; 1.0638x over previous
import functools

import jax
import jax.numpy as jnp
from jax import lax
from jax.experimental import pallas as pl
from jax.experimental.pallas import tpu as pltpu
from jax.experimental.pallas import tpu_sc as plsc

_UNPOOL = (0, 0, 1, 1, 2, 3, 3, 4, 5, 5, 6, 7, 7, 8, 8, 9, 9, 10, 10, 11, 11, 11)

_FANOUT = tuple(
    tuple(j for j, src in enumerate(_UNPOOL) if src == jp)
    for jp in range(_UNPOOL[-1] + 1)
)


def kernel(x_p):
    B, JP, C, F = x_p.shape
    NJ = len(_UNPOOL)
    D = C * F
    x2 = x_p.reshape(B, JP, D)

    info = plsc.get_sparse_core_info()
    NC, NS = info.num_cores, info.num_subcores
    NW = NC * NS
    bpw = B // NW

    mesh = plsc.VectorSubcoreMesh(core_axis_name="c", subcore_axis_name="s")

    NBUF = 4
    LAG = NBUF - 1

    @functools.partial(
        pl.kernel,
        mesh=mesh,
        out_type=jax.ShapeDtypeStruct((B, NJ, D), jnp.float32),
        scratch_types=[
            pltpu.VMEM((NBUF, D), jnp.float32),
            pltpu.SemaphoreType.DMA((NBUF,)),
            pltpu.SemaphoreType.DMA((NBUF,)),
        ],
    )
    def unpool_sc(x_hbm, out_hbm, bufs, gsems, ssems):
        wid = lax.axis_index("s") * NC + lax.axis_index("c")
        rows = [(bi, jp) for bi in range(bpw) for jp in range(JP)]
        gpend = [None] * NBUF
        spend = [[] for _ in range(NBUF)]
        for s in range(len(rows) + LAG):
            if s < len(rows):
                p = s % NBUF
                for c in spend[p]:
                    c.wait()
                spend[p] = []
                bi, jp = rows[s]
                b = wid * bpw + bi
                gpend[p] = pltpu.async_copy(x_hbm.at[b, jp], bufs.at[p], gsems.at[p])
            t = s - LAG
            if t >= 0:
                q = t % NBUF
                gpend[q].wait()
                bi, jp = rows[t]
                b = wid * bpw + bi
                for j in _FANOUT[jp]:
                    spend[q].append(
                        pltpu.async_copy(bufs.at[q], out_hbm.at[b, j], ssems.at[q])
                    )
        for q in range(NBUF):
            for c in spend[q]:
                c.wait()

    return unpool_sc(x2).reshape(B, NJ, C, F)


# speedup vs baseline: 5.0106x; 4.7099x over previous
import functools

import jax
import jax.numpy as jnp
from jax import lax
from jax.experimental import pallas as pl
from jax.experimental.pallas import tpu as pltpu
from jax.experimental.pallas import tpu_sc as plsc

_UNPOOL = (0, 0, 1, 1, 2, 3, 3, 4, 5, 5, 6, 7, 7, 8, 8, 9, 9, 10, 10, 11, 11, 11)

_FANOUT = tuple(
    tuple(j for j, src in enumerate(_UNPOOL) if src == jp)
    for jp in range(_UNPOOL[-1] + 1)
)


def kernel(x_p):
    B, JP, C, F = x_p.shape
    NJ = len(_UNPOOL)

    x_t = jnp.swapaxes(x_p, 2, 3)

    info = plsc.get_sparse_core_info()
    NC, NS = info.num_cores, info.num_subcores
    NW = NC * NS
    bpw = B // NW

    mesh = plsc.VectorSubcoreMesh(core_axis_name="c", subcore_axis_name="s")

    CHUNK = 1
    NBUF = 6
    LAG = NBUF - 1
    NCH = JP // CHUNK

    @functools.partial(
        pl.kernel,
        mesh=mesh,
        out_type=jax.ShapeDtypeStruct((B, NJ, F, C), jnp.float32),
        scratch_types=[
            pltpu.VMEM((NBUF, CHUNK, F, C), jnp.float32),
            pltpu.SemaphoreType.DMA((NBUF,)),
            pltpu.SemaphoreType.DMA((NBUF,)),
        ],
        compiler_params=pltpu.CompilerParams(use_tc_tiling_on_sc=True),
    )
    def unpool_sc(x_hbm, out_hbm, bufs, gsems, ssems):
        wid = lax.axis_index("s") * NC + lax.axis_index("c")
        steps = [(bi, ch) for bi in range(bpw) for ch in range(NCH)]
        gpend = [None] * NBUF
        spend = [[] for _ in range(NBUF)]
        for s in range(len(steps) + LAG):
            if s < len(steps):
                p = s % NBUF
                for c in spend[p]:
                    c.wait()
                spend[p] = []
                bi, ch = steps[s]
                b = wid * bpw + bi
                gpend[p] = pltpu.async_copy(
                    x_hbm.at[b, pl.ds(ch * CHUNK, CHUNK)], bufs.at[p], gsems.at[p]
                )
            t = s - LAG
            if t >= 0:
                q = t % NBUF
                gpend[q].wait()
                bi, ch = steps[t]
                b = wid * bpw + bi
                for k in range(CHUNK):
                    for j in _FANOUT[ch * CHUNK + k]:
                        spend[q].append(
                            pltpu.async_copy(
                                bufs.at[q, k], out_hbm.at[b, j], ssems.at[q]
                            )
                        )
        for q in range(NBUF):
            for c in spend[q]:
                c.wait()

    return jnp.swapaxes(unpool_sc(x_t), 2, 3)


# speedup vs baseline: 5.1260x; 1.0230x over previous
import functools

import jax
import jax.numpy as jnp
from jax import lax
from jax.experimental import pallas as pl
from jax.experimental.pallas import tpu as pltpu
from jax.experimental.pallas import tpu_sc as plsc

_UNPOOL = (0, 0, 1, 1, 2, 3, 3, 4, 5, 5, 6, 7, 7, 8, 8, 9, 9, 10, 10, 11, 11, 11)

_FANOUT = tuple(
    tuple(j for j, src in enumerate(_UNPOOL) if src == jp)
    for jp in range(_UNPOOL[-1] + 1)
)


def kernel(x_p):
    B, JP, C, F = x_p.shape
    NJ = len(_UNPOOL)

    x_t = jnp.swapaxes(x_p, 2, 3)

    info = plsc.get_sparse_core_info()
    NC, NS = info.num_cores, info.num_subcores
    NW = NC * NS
    bpw = B // NW

    mesh = plsc.VectorSubcoreMesh(core_axis_name="c", subcore_axis_name="s")

    CHUNK = 2
    NBUF = 4
    LAG = NBUF - 1
    NCH = JP // CHUNK

    @functools.partial(
        pl.kernel,
        mesh=mesh,
        out_type=jax.ShapeDtypeStruct((B, NJ, F, C), jnp.float32),
        scratch_types=[
            pltpu.VMEM((NBUF, CHUNK, F, C), jnp.float32),
            pltpu.SemaphoreType.DMA((NBUF,)),
            pltpu.SemaphoreType.DMA((NBUF,)),
        ],
        compiler_params=pltpu.CompilerParams(use_tc_tiling_on_sc=True),
    )
    def unpool_sc(x_hbm, out_hbm, bufs, gsems, ssems):
        wid = lax.axis_index("s") * NC + lax.axis_index("c")
        steps = [(bi, ch) for bi in range(bpw) for ch in range(NCH)]
        gpend = [None] * NBUF
        spend = [[] for _ in range(NBUF)]
        for s in range(len(steps) + LAG):
            if s < len(steps):
                p = s % NBUF
                for c in spend[p]:
                    c.wait()
                spend[p] = []
                bi, ch = steps[s]
                b = wid * bpw + bi
                gpend[p] = pltpu.async_copy(
                    x_hbm.at[b, pl.ds(ch * CHUNK, CHUNK)], bufs.at[p], gsems.at[p]
                )
            t = s - LAG
            if t >= 0:
                q = t % NBUF
                gpend[q].wait()
                bi, ch = steps[t]
                b = wid * bpw + bi
                for k in range(CHUNK):
                    for j in _FANOUT[ch * CHUNK + k]:
                        spend[q].append(
                            pltpu.async_copy(
                                bufs.at[q, k], out_hbm.at[b, j], ssems.at[q]
                            )
                        )
        for q in range(NBUF):
            for c in spend[q]:
                c.wait()

    return jnp.swapaxes(unpool_sc(x_t), 2, 3)
